# baseline (speedup 1.0000x reference)
import functools

import jax
import jax.numpy as jnp
from jax import lax
from jax.experimental import pallas as pl
from jax.experimental.pallas import tpu as pltpu
from jax.experimental.pallas import tpu_sc as plsc

V, D = 10000, 128
E = 320000
NC, NS, L = 2, 16, 16
NR = 8
NPAIR = 8
EPC = E // NC
EPG = EPC // 2
CK = 2000
NCK = NR
S = CK * NCK
NSUP = EPG // S
NCKT = NSUP * NCK
NG = CK // L
SP = 2048
NWV = 4


def _decode(tab_p, xy):
    mesh = plsc.VectorSubcoreMesh(core_axis_name="c", subcore_axis_name="s")

    @functools.partial(
        pl.kernel,
        out_type=jax.ShapeDtypeStruct((E,), jnp.float32),
        mesh=mesh,
        scratch_types=[
            pltpu.VMEM((NPAIR * V,), jnp.int32),
            pltpu.VMEM((CK,), jnp.int32),
            pltpu.VMEM((CK,), jnp.int32),
            pltpu.VMEM((SP,), jnp.float32),
            pltpu.VMEM((SP,), jnp.float32),
            pltpu.VMEM((SP,), jnp.float32),
            pltpu.VMEM((SP,), jnp.float32),
            pltpu.VMEM((CK,), jnp.float32),
            pltpu.VMEM_SHARED((NS, NR, SP), jnp.float32),
            pltpu.SemaphoreType.DMA,
            pltpu.SemaphoreType.DMA,
            pltpu.SemaphoreType.DMA,
            pltpu.SemaphoreType.DMA,
        ],
        compiler_params=pltpu.CompilerParams(needs_layout_passes=False),
    )
    def k(tab_h, xy_h, out_h, shard, xyc0, xyc1, pb0, pb1,
          rb0, rb1, outb, stage, sem_in, sem_st, sem_rd, sem_rd2):
        score = lax.axis_index("c")
        sid = lax.axis_index("s")
        gi = sid // NR
        ri = sid % NR
        ebase = pl.multiple_of(score * EPC + gi * EPG, 8)
        rbs = (rb0, rb1)

        pltpu.sync_copy(tab_h.at[pl.ds(ri * (NPAIR * V), NPAIR * V)], shard)

        pltpu.async_copy(xy_h.at[pl.ds(ebase, CK)], xyc0, sem_in)

        def phase1_chunk(xyc_cur, xyc_nxt, pb_cur, ch, ci):
            pltpu.make_async_copy(
                xy_h.at[pl.ds(ebase, CK)], xyc_cur, sem_in).wait()

            @pl.when(ci + 1 < NCKT)
            def _():
                nbase = ebase + (ci + 1) * CK
                pltpu.async_copy(xy_h.at[pl.ds(nbase, CK)], xyc_nxt, sem_in)

            @pl.when(ch >= 2)
            def _():
                pltpu.make_async_copy(
                    pb_cur, stage.at[sid].at[0], sem_st).wait()

            def group_body(g, carry):
                w = xyc_cur[pl.ds(g * L, L)]
                xa = w & 0xFFFF
                ya = lax.shift_right_logical(w, 16)
                acc = [jnp.zeros((L,), jnp.float32) for _ in range(4)]
                for p in range(NPAIR):
                    xw = plsc.load_gather(shard, [xa + (p * V)])
                    yw = plsc.load_gather(shard, [ya + (p * V)])
                    prod = (plsc.bitcast(xw, jnp.bfloat16)
                            * plsc.bitcast(yw, jnp.bfloat16))
                    p0, p1 = plsc.unpack(
                        prod, format=plsc.PackFormat.INTERLEAVED)
                    acc[p % 4] = acc[p % 4] + p0
                    acc[p % 4] = acc[p % 4] + p1
                pb_cur[pl.ds(g * L, L)] = (
                    (acc[0] + acc[1]) + (acc[2] + acc[3]))
                return carry

            lax.fori_loop(0, NG, group_body, 0, unroll=2)

            pltpu.async_copy(pb_cur, stage.at[sid].at[ch], sem_st)

        def sup_body(s, carry):
            sbase = ebase + s * S

            def chunk_body(ch, carry):
                ci = s * NCK + ch

                @pl.when((ch & 1) == 0)
                def _():
                    phase1_chunk(xyc0, xyc1, pb0, ch, ci)

                @pl.when((ch & 1) == 1)
                def _():
                    phase1_chunk(xyc1, xyc0, pb1, ch, ci)

                return carry

            lax.fori_loop(0, NCK, chunk_body, 0)

            pltpu.make_async_copy(pb0, stage.at[sid].at[0], sem_st).wait()
            pltpu.make_async_copy(pb1, stage.at[sid].at[0], sem_st).wait()
            plsc.subcore_barrier()

            wbs = ((rb0, rb1), (pb0, pb1))
            sems = (sem_rd, sem_rd2)

            def fire(w):
                for i in range(NR // NWV):
                    pltpu.async_copy(
                        stage.at[gi * NR + w * (NR // NWV) + i].at[ri],
                        wbs[w & 1][i], sems[w & 1])

            fire(0)
            fire(1)
            for w in range(NWV):
                for i in range(NR // NWV):
                    pltpu.make_async_copy(
                        stage.at[0].at[0], wbs[w & 1][i],
                        sems[w & 1]).wait()

                def red_body(g, carry):
                    o = g * L
                    a = wbs[w & 1][0][pl.ds(o, L)]
                    for i in range(1, NR // NWV):
                        a = a + wbs[w & 1][i][pl.ds(o, L)]
                    if w > 0:
                        a = a + outb[pl.ds(o, L)]
                    outb[pl.ds(o, L)] = a
                    return carry

                lax.fori_loop(0, NG, red_body, 0)
                if w + 2 < NWV:
                    fire(w + 2)

            def sig_body(g, carry):
                o = g * L
                a = outb[pl.ds(o, L)]
                outb[pl.ds(o, L)] = 1.0 / (1.0 + jnp.exp(-a))
                return carry

            lax.fori_loop(0, NG, sig_body, 0)
            goff = pl.multiple_of(sbase + ri * CK, 8)
            pltpu.sync_copy(outb, out_h.at[pl.ds(goff, CK)])

            plsc.subcore_barrier()
            return carry

        lax.fori_loop(0, NSUP, sup_body, 0)

    return k(tab_p, xy)


def kernel(inputs, x_idx, y_idx):
    bf = inputs.astype(jnp.bfloat16).reshape(V, NR, NPAIR, 2)
    packed = jax.lax.bitcast_convert_type(bf, jnp.int32)
    tab_p = jnp.transpose(packed, (1, 2, 0)).reshape(NR * NPAIR * V)
    xy = x_idx.astype(jnp.int32) | (y_idx.astype(jnp.int32) << 16)
    return _decode(tab_p, xy)


# speedup vs baseline: 1.2103x; 1.2103x over previous
import functools

import jax
import jax.numpy as jnp
from jax import lax
from jax.experimental import pallas as pl
from jax.experimental.pallas import tpu as pltpu
from jax.experimental.pallas import tpu_sc as plsc

V, D = 10000, 128
E = 320000
NC, NS, L = 2, 16, 16
NR = 8
NPAIR = 8
EPC = E // NC
EPG = EPC // 2
CK = 2000
NCK = NR
S = CK * NCK
NSUP = EPG // S
NCKT = NSUP * NCK
NG = CK // L
SP = 2048
NWV = 4


def _decode(tab_p, xy):
    mesh = plsc.VectorSubcoreMesh(core_axis_name="c", subcore_axis_name="s")

    @functools.partial(
        pl.kernel,
        out_type=jax.ShapeDtypeStruct((E,), jnp.float32),
        mesh=mesh,
        scratch_types=[
            pltpu.VMEM((NPAIR * V,), jnp.int32),
            pltpu.VMEM((CK,), jnp.int32),
            pltpu.VMEM((CK,), jnp.int32),
            pltpu.VMEM((SP,), jnp.float32),
            pltpu.VMEM((SP,), jnp.float32),
            pltpu.VMEM((SP,), jnp.float32),
            pltpu.VMEM((SP,), jnp.float32),
            pltpu.VMEM((CK,), jnp.float32),
            pltpu.VMEM_SHARED((NS, NR, SP), jnp.float32),
            pltpu.SemaphoreType.DMA,
            pltpu.SemaphoreType.DMA,
            pltpu.SemaphoreType.DMA,
            pltpu.SemaphoreType.DMA,
        ],
        compiler_params=pltpu.CompilerParams(needs_layout_passes=False),
    )
    def k(tab_h, xy_h, out_h, shard, xyc0, xyc1, pb0, pb1,
          rb0, rb1, outb, stage, sem_in, sem_st, sem_rd, sem_rd2):
        score = lax.axis_index("c")
        sid = lax.axis_index("s")
        gi = sid // NR
        ri = sid % NR
        ebase = pl.multiple_of(score * EPC + gi * EPG, 8)
        rbs = (rb0, rb1)

        pltpu.sync_copy(tab_h.at[pl.ds(ri * (NPAIR * V), NPAIR * V)], shard)

        pltpu.async_copy(xy_h.at[pl.ds(ebase, CK)], xyc0, sem_in)

        def phase1_chunk(xyc_cur, xyc_nxt, pb_cur, ch, ci):
            pltpu.make_async_copy(
                xy_h.at[pl.ds(ebase, CK)], xyc_cur, sem_in).wait()

            @pl.when(ci + 1 < NCKT)
            def _():
                nbase = ebase + (ci + 1) * CK
                pltpu.async_copy(xy_h.at[pl.ds(nbase, CK)], xyc_nxt, sem_in)

            @pl.when(ch >= 2)
            def _():
                pltpu.make_async_copy(
                    pb_cur, stage.at[sid].at[0], sem_st).wait()

            def group_body(g, carry):
                w = xyc_cur[pl.ds(g * L, L)]
                xa = w & 0xFFFF
                ya = lax.shift_right_logical(w, 16)
                acc = []
                for p in range(NPAIR):
                    xw = plsc.load_gather(shard, [xa + (p * V)])
                    yw = plsc.load_gather(shard, [ya + (p * V)])
                    prod = (plsc.bitcast(xw, jnp.bfloat16)
                            * plsc.bitcast(yw, jnp.bfloat16))
                    p0, p1 = plsc.unpack(
                        prod, format=plsc.PackFormat.INTERLEAVED)
                    acc.append(p0 + p1)
                a01 = (acc[0] + acc[1]) + (acc[2] + acc[3])
                a23 = (acc[4] + acc[5]) + (acc[6] + acc[7])
                pb_cur[pl.ds(g * L, L)] = a01 + a23
                return carry

            lax.fori_loop(0, NG, group_body, 0)

            pltpu.async_copy(pb_cur, stage.at[sid].at[ch], sem_st)

        def sup_body(s, carry):
            sbase = ebase + s * S

            def chunk_body(ch, carry):
                ci = s * NCK + ch

                @pl.when((ch & 1) == 0)
                def _():
                    phase1_chunk(xyc0, xyc1, pb0, ch, ci)

                @pl.when((ch & 1) == 1)
                def _():
                    phase1_chunk(xyc1, xyc0, pb1, ch, ci)

                return carry

            lax.fori_loop(0, NCK, chunk_body, 0)

            pltpu.make_async_copy(pb0, stage.at[sid].at[0], sem_st).wait()
            pltpu.make_async_copy(pb1, stage.at[sid].at[0], sem_st).wait()
            plsc.subcore_barrier()

            wbs = ((rb0, rb1), (pb0, pb1))
            sems = (sem_rd, sem_rd2)

            def fire(w):
                for i in range(NR // NWV):
                    pltpu.async_copy(
                        stage.at[gi * NR + w * (NR // NWV) + i].at[ri],
                        wbs[w & 1][i], sems[w & 1])

            fire(0)
            fire(1)
            for w in range(NWV):
                for i in range(NR // NWV):
                    pltpu.make_async_copy(
                        stage.at[0].at[0], wbs[w & 1][i],
                        sems[w & 1]).wait()

                def red_body(g, carry):
                    o = g * L
                    a = wbs[w & 1][0][pl.ds(o, L)]
                    for i in range(1, NR // NWV):
                        a = a + wbs[w & 1][i][pl.ds(o, L)]
                    if w > 0:
                        a = a + outb[pl.ds(o, L)]
                    outb[pl.ds(o, L)] = a
                    return carry

                lax.fori_loop(0, NG, red_body, 0)
                if w + 2 < NWV:
                    fire(w + 2)

            def sig_body(g, carry):
                o = g * L
                a = outb[pl.ds(o, L)]
                outb[pl.ds(o, L)] = 1.0 / (1.0 + jnp.exp(-a))
                return carry

            lax.fori_loop(0, NG, sig_body, 0)
            goff = pl.multiple_of(sbase + ri * CK, 8)
            pltpu.sync_copy(outb, out_h.at[pl.ds(goff, CK)])

            plsc.subcore_barrier()
            return carry

        lax.fori_loop(0, NSUP, sup_body, 0)

    return k(tab_p, xy)


def kernel(inputs, x_idx, y_idx):
    bf = inputs.astype(jnp.bfloat16).reshape(V, NR, NPAIR, 2)
    packed = jax.lax.bitcast_convert_type(bf, jnp.int32)
    tab_p = jnp.transpose(packed, (1, 2, 0)).reshape(NR * NPAIR * V)
    xy = x_idx.astype(jnp.int32) | (y_idx.astype(jnp.int32) << 16)
    return _decode(tab_p, xy)
